# Initial kernel scaffold; baseline (speedup 1.0000x reference)
import jax
import jax.numpy as jnp
from jax.experimental import pallas as pl


def kernel(x, edge_index, edge_type, edge_weight, W_lin, b_lin, W_self, b_self):
    raise NotImplementedError("write your pallas kernel here")


# speedup vs baseline: 9.5429x; 9.5429x over previous
import functools

import jax
import jax.numpy as jnp
from jax import lax
from jax.experimental import pallas as pl
from jax.experimental.pallas import tpu as pltpu
from jax.experimental.pallas import tpu_sc as plsc

_N = 10000
_R = 8
_D = 128
_OUT = 128
_E = 320000
_NR = _N * _R

_NC = 2
_NS = 16
_NW = _NC * _NS

_GRP = 128
_GPC = 6
_CH = _GRP * _GPC

_DEG_CHUNKS = 26
_DEG_MAIN = _DEG_CHUNKS * _CH
_DEG_COVERED = _DEG_MAIN * _NS

_ROW_CHUNKS = 13
_ROW_MAIN = _ROW_CHUNKS * _CH
_ROW_COVERED = _ROW_MAIN * _NW
_TAIL_BASE = 319488
_TAIL = 512

_RPT = _N // _NS


def _sc_body(src_hbm, dst_hbm, rel_hbm, w_hbm, xw_hbm, out_hbm,
             z2, zflat, st_src, st_dst, st_rel, st_w,
             seg2d, gidx2d, dst2d, degv, ew_st, rows, acc_sh, deg_sh, sem):
    c = lax.axis_index("c")
    s = lax.axis_index("s")
    wid = s * _NC + c

    def _zz2(i, _):
        z2[i // 8, pl.ds((i % 8) * 16, 16)] = jnp.zeros((16,), jnp.float32)
        return _
    lax.fori_loop(0, 1024, _zz2, None)

    def _zzf(i, _):
        zflat[pl.ds(i * 16, 16)] = jnp.zeros((16,), jnp.float32)
        return _
    lax.fori_loop(0, 320, _zzf, None)

    def _zacc(r0, nrows):
        pltpu.sync_copy(z2.at[pl.ds(0, nrows)], acc_sh.at[pl.ds(r0, nrows)])

    @pl.when(s < 15)
    def _():
        for k in range(4):
            _zacc(s * 624 + k * 128, 128)
        _zacc(s * 624 + 512, 112)

    @pl.when(s == 15)
    def _():
        for k in range(5):
            _zacc(9360 + k * 128, 128)

    @pl.when(s < 15)
    def _():
        pltpu.sync_copy(zflat, deg_sh.at[pl.ds(s * 5120, 5120)])

    @pl.when(s == 15)
    def _():
        pltpu.sync_copy(zflat.at[pl.ds(0, 3200)], deg_sh.at[pl.ds(76800, 3200)])

    plsc.subcore_barrier()

    def _deg_chunk(base, ngrp):
        n = ngrp * _GRP
        pltpu.sync_copy(dst_hbm.at[pl.ds(base, n)], st_dst.at[pl.ds(0, n)])
        pltpu.sync_copy(rel_hbm.at[pl.ds(base, n)], st_rel.at[pl.ds(0, n)])
        pltpu.sync_copy(w_hbm.at[pl.ds(base, n)], st_w.at[pl.ds(0, n)])

        def _mk(g, _):
            d16 = st_dst[pl.ds(g * 16, 16)]
            t16 = st_rel[pl.ds(g * 16, 16)]
            seg2d[g // 8, pl.ds((g % 8) * 16, 16)] = d16 * _R + t16
            return _
        lax.fori_loop(0, ngrp * 8, _mk, None)
        for g in range(ngrp):
            pltpu.sync_copy(st_w.at[pl.ds(g * _GRP, _GRP)],
                            deg_sh.at[seg2d.at[g]], add=True)

    def _deg_loop(k, _):
        _deg_chunk(s * _DEG_MAIN + k * _CH, _GPC)
        return _
    lax.fori_loop(0, _DEG_CHUNKS, _deg_loop, None)

    @pl.when(s == 0)
    def _():
        _deg_chunk(_TAIL_BASE, 4)

    plsc.subcore_barrier()

    def _row_chunk(base, ngrp):
        n = ngrp * _GRP
        pltpu.sync_copy(src_hbm.at[pl.ds(base, n)], st_src.at[pl.ds(0, n)])
        pltpu.sync_copy(dst_hbm.at[pl.ds(base, n)], st_dst.at[pl.ds(0, n)])
        pltpu.sync_copy(rel_hbm.at[pl.ds(base, n)], st_rel.at[pl.ds(0, n)])
        pltpu.sync_copy(w_hbm.at[pl.ds(base, n)], st_w.at[pl.ds(0, n)])

        def _mk(g, _):
            s16 = st_src[pl.ds(g * 16, 16)]
            d16 = st_dst[pl.ds(g * 16, 16)]
            t16 = st_rel[pl.ds(g * 16, 16)]
            r = g // 8
            col = (g % 8) * 16
            gidx2d[r, pl.ds(col, 16)] = s16 * _R + t16
            seg2d[r, pl.ds(col, 16)] = d16 * _R + t16
            dst2d[r, pl.ds(col, 16)] = d16
            return _
        lax.fori_loop(0, ngrp * 8, _mk, None)

        for g in range(ngrp):
            pltpu.async_copy(deg_sh.at[seg2d.at[g]],
                             degv.at[pl.ds(g * _GRP, _GRP)], sem).wait()

        def _ew(k, _):
            ew_st[pl.ds(k * 16, 16)] = (st_w[pl.ds(k * 16, 16)]
                                        / degv[pl.ds(k * 16, 16)])
            return _
        lax.fori_loop(0, ngrp * 8, _ew, None)

        for g in range(ngrp):
            pltpu.async_copy(xw_hbm.at[gidx2d.at[g]], rows, sem).wait()

            def _scale(i, _):
                ewv = ew_st[pl.ds(g * _GRP + i * 16, 16)]
                for e in range(16):
                    sc = ewv[e]
                    r = i * 16 + e
                    for j in range(8):
                        rows[r, pl.ds(j * 16, 16)] = (
                            rows[r, pl.ds(j * 16, 16)] * sc)
                return _
            lax.fori_loop(0, 8, _scale, None)
            pltpu.sync_copy(rows, acc_sh.at[dst2d.at[g]], add=True)

    def _row_loop(k, _):
        _row_chunk(wid * _ROW_MAIN + k * _CH, _GPC)
        return _
    lax.fori_loop(0, _ROW_CHUNKS, _row_loop, None)

    @pl.when(wid == 0)
    def _():
        _row_chunk(_TAIL_BASE, 4)

    plsc.subcore_barrier()

    def _dump(r0, nrows):
        pltpu.sync_copy(acc_sh.at[pl.ds(r0, nrows)], rows.at[pl.ds(0, nrows)])
        pltpu.sync_copy(rows.at[pl.ds(0, nrows)], out_hbm.at[c, pl.ds(r0, nrows)])

    @pl.when(s < 15)
    def _():
        for k in range(4):
            _dump(s * 624 + k * 128, 128)
        _dump(s * 624 + 512, 112)

    @pl.when(s == 15)
    def _():
        for k in range(5):
            _dump(9360 + k * 128, 128)


def _mm_body(x_ref, w_ref, o_ref):
    o_ref[...] = jnp.dot(x_ref[...], w_ref[...],
                         preferred_element_type=jnp.float32)


def _comb_body(p_ref, x_ref, ws_ref, b_ref, o_ref):
    acc = p_ref[0] + p_ref[1]
    acc = acc + jnp.dot(x_ref[...], ws_ref[...],
                        preferred_element_type=jnp.float32)
    o_ref[...] = jnp.maximum(acc + b_ref[...], 0.0)


def kernel(x, edge_index, edge_type, edge_weight, W_lin, b_lin, W_self, b_self):
    src = edge_index[0]
    dst = edge_index[1]

    Wt = jnp.transpose(W_lin.reshape(_OUT, _R, _D), (2, 1, 0)).reshape(_D, _R * _OUT)

    xw = pl.pallas_call(
        _mm_body,
        grid=(10,),
        in_specs=[
            pl.BlockSpec((1000, _D), lambda i: (i, 0)),
            pl.BlockSpec((_D, _R * _OUT), lambda i: (0, 0)),
        ],
        out_specs=pl.BlockSpec((1000, _R * _OUT), lambda i: (i, 0)),
        out_shape=jax.ShapeDtypeStruct((_N, _R * _OUT), jnp.float32),
    )(x, Wt)
    xw = xw.reshape(_NR, _OUT)

    mesh = plsc.VectorSubcoreMesh(core_axis_name="c", subcore_axis_name="s",
                                  num_cores=_NC, num_subcores=_NS)
    sc_fn = pl.kernel(
        _sc_body,
        out_type=jax.ShapeDtypeStruct((_NC, _N, _OUT), jnp.float32),
        mesh=mesh,
        scratch_types=[
            pltpu.VMEM((128, 128), jnp.float32),
            pltpu.VMEM((5120,), jnp.float32),
            pltpu.VMEM((_CH,), jnp.int32),
            pltpu.VMEM((_CH,), jnp.int32),
            pltpu.VMEM((_CH,), jnp.int32),
            pltpu.VMEM((_CH,), jnp.float32),
            pltpu.VMEM((_GPC, _GRP), jnp.int32),
            pltpu.VMEM((_GPC, _GRP), jnp.int32),
            pltpu.VMEM((_GPC, _GRP), jnp.int32),
            pltpu.VMEM((_CH,), jnp.float32),
            pltpu.VMEM((_CH,), jnp.float32),
            pltpu.VMEM((_GRP, _OUT), jnp.float32),
            pltpu.VMEM_SHARED((_N, _OUT), jnp.float32),
            pltpu.VMEM_SHARED((_NR,), jnp.float32),
            pltpu.SemaphoreType.DMA,
        ],
    )
    parts = sc_fn(src, dst, edge_type, edge_weight, xw)

    bias = (b_lin + b_self).reshape(1, _OUT)
    Wst = W_self.T

    out = pl.pallas_call(
        _comb_body,
        grid=(10,),
        in_specs=[
            pl.BlockSpec((_NC, 1000, _OUT), lambda i: (0, i, 0)),
            pl.BlockSpec((1000, _D), lambda i: (i, 0)),
            pl.BlockSpec((_D, _OUT), lambda i: (0, 0)),
            pl.BlockSpec((1, _OUT), lambda i: (0, 0)),
        ],
        out_specs=pl.BlockSpec((1000, _OUT), lambda i: (i, 0)),
        out_shape=jax.ShapeDtypeStruct((_N, _OUT), jnp.float32),
    )(parts, x, Wst, bias)
    return out


# speedup vs baseline: 12.4130x; 1.3008x over previous
import functools

import jax
import jax.numpy as jnp
from jax import lax
from jax.experimental import pallas as pl
from jax.experimental.pallas import tpu as pltpu
from jax.experimental.pallas import tpu_sc as plsc

_N = 10000
_R = 8
_D = 128
_OUT = 128
_E = 320000
_NR = _N * _R

_NC = 2
_NS = 16
_NW = _NC * _NS

_GRP = 128
_GPC = 6
_CH = _GRP * _GPC

_DEG_CHUNKS = 26
_DEG_MAIN = _DEG_CHUNKS * _CH
_DEG_COVERED = _DEG_MAIN * _NS

_ROW_CHUNKS = 13
_ROW_MAIN = _ROW_CHUNKS * _CH
_ROW_COVERED = _ROW_MAIN * _NW
_TAIL_BASE = 319488
_TAIL = 512

_RPT = _N // _NS


def _sc_body(src_hbm, dst_hbm, rel_hbm, w_hbm, xw_hbm, out_hbm,
             zflat, st_src, st_dst, st_rel, st_w,
             seg2d, gidx2d, dst2d, degv, ew_st, rows, rows_b, acc_sh, deg_sh,
             sem, sem2):
    c = lax.axis_index("c")
    s = lax.axis_index("s")
    wid = s * _NC + c

    def _zrows(i, _):
        rows[i // 8, pl.ds((i % 8) * 16, 16)] = jnp.zeros((16,), jnp.float32)
        return _
    lax.fori_loop(0, 1024, _zrows, None)

    def _zzf(i, _):
        zflat[pl.ds(i * 16, 16)] = jnp.zeros((16,), jnp.float32)
        return _
    lax.fori_loop(0, 80, _zzf, None)

    def _zacc(r0, nrows):
        pltpu.sync_copy(rows.at[pl.ds(0, nrows)], acc_sh.at[pl.ds(r0, nrows)])

    @pl.when(s < 15)
    def _():
        for k in range(4):
            _zacc(s * 624 + k * 128, 128)
        _zacc(s * 624 + 512, 112)

    @pl.when(s == 15)
    def _():
        for k in range(5):
            _zacc(9360 + k * 128, 128)

    @pl.when(s < 15)
    def _():
        for k in range(4):
            pltpu.sync_copy(zflat, deg_sh.at[pl.ds(s * 5120 + k * 1280, 1280)])

    @pl.when(s == 15)
    def _():
        for k in range(2):
            pltpu.sync_copy(zflat, deg_sh.at[pl.ds(76800 + k * 1280, 1280)])
        pltpu.sync_copy(zflat.at[pl.ds(0, 640)], deg_sh.at[pl.ds(79360, 640)])

    plsc.subcore_barrier()

    def _deg_chunk(base, ngrp):
        n = ngrp * _GRP
        ds_ = [pltpu.async_copy(dst_hbm.at[pl.ds(base, n)],
                                st_dst.at[pl.ds(0, n)], sem),
               pltpu.async_copy(rel_hbm.at[pl.ds(base, n)],
                                st_rel.at[pl.ds(0, n)], sem),
               pltpu.async_copy(w_hbm.at[pl.ds(base, n)],
                                st_w.at[pl.ds(0, n)], sem)]
        for d in ds_:
            d.wait()

        def _mk(g, _):
            d16 = st_dst[pl.ds(g * 16, 16)]
            t16 = st_rel[pl.ds(g * 16, 16)]
            seg2d[g // 8, pl.ds((g % 8) * 16, 16)] = d16 * _R + t16
            return _
        lax.fori_loop(0, ngrp * 8, _mk, None)
        ds_ = [pltpu.async_copy(st_w.at[pl.ds(g * _GRP, _GRP)],
                                deg_sh.at[seg2d.at[g]], sem2, add=True)
               for g in range(ngrp)]
        for d in ds_:
            d.wait()

    def _deg_loop(k, _):
        _deg_chunk(s * _DEG_MAIN + k * _CH, _GPC)
        return _
    lax.fori_loop(0, _DEG_CHUNKS, _deg_loop, None)

    @pl.when(s == 0)
    def _():
        _deg_chunk(_TAIL_BASE, 4)

    plsc.subcore_barrier()

    def _row_chunk(base, ngrp):
        n = ngrp * _GRP
        ds_ = [pltpu.async_copy(src_hbm.at[pl.ds(base, n)],
                                st_src.at[pl.ds(0, n)], sem),
               pltpu.async_copy(dst_hbm.at[pl.ds(base, n)],
                                st_dst.at[pl.ds(0, n)], sem),
               pltpu.async_copy(rel_hbm.at[pl.ds(base, n)],
                                st_rel.at[pl.ds(0, n)], sem),
               pltpu.async_copy(w_hbm.at[pl.ds(base, n)],
                                st_w.at[pl.ds(0, n)], sem)]
        for d in ds_:
            d.wait()

        def _mk(g, _):
            s16 = st_src[pl.ds(g * 16, 16)]
            d16 = st_dst[pl.ds(g * 16, 16)]
            t16 = st_rel[pl.ds(g * 16, 16)]
            r = g // 8
            col = (g % 8) * 16
            gidx2d[r, pl.ds(col, 16)] = s16 * _R + t16
            seg2d[r, pl.ds(col, 16)] = d16 * _R + t16
            dst2d[r, pl.ds(col, 16)] = d16
            return _
        lax.fori_loop(0, ngrp * 8, _mk, None)

        ds_ = [pltpu.async_copy(deg_sh.at[seg2d.at[g]],
                                degv.at[pl.ds(g * _GRP, _GRP)], sem)
               for g in range(ngrp)]
        for d in ds_:
            d.wait()

        def _ew(k, _):
            ew_st[pl.ds(k * 16, 16)] = (st_w[pl.ds(k * 16, 16)]
                                        / degv[pl.ds(k * 16, 16)])
            return _
        lax.fori_loop(0, ngrp * 8, _ew, None)

        def _scale(buf, g):
            def body(i, _):
                ewv = ew_st[pl.ds(g * _GRP + i * 16, 16)]
                for e in range(16):
                    sc = ewv[e]
                    r = i * 16 + e
                    for j in range(8):
                        buf[r, pl.ds(j * 16, 16)] = (
                            buf[r, pl.ds(j * 16, 16)] * sc)
                return _
            lax.fori_loop(0, 8, body, None)

        bufs = (rows, rows_b)
        d_g = {0: pltpu.async_copy(xw_hbm.at[gidx2d.at[0]], bufs[0], sem)}
        d_s = {}
        for g in range(ngrp):
            p = g % 2
            d_g[g].wait()
            _scale(bufs[p], g)
            d_s[g] = pltpu.async_copy(bufs[p], acc_sh.at[dst2d.at[g]],
                                      sem2, add=True)
            if g + 1 < ngrp:
                if g >= 1:
                    d_s[g - 1].wait()
                d_g[g + 1] = pltpu.async_copy(xw_hbm.at[gidx2d.at[g + 1]],
                                              bufs[(g + 1) % 2], sem)
        if ngrp >= 2:
            d_s[ngrp - 2].wait()
        d_s[ngrp - 1].wait()

    def _row_loop(k, _):
        _row_chunk(wid * _ROW_MAIN + k * _CH, _GPC)
        return _
    lax.fori_loop(0, _ROW_CHUNKS, _row_loop, None)

    @pl.when(wid == 0)
    def _():
        _row_chunk(_TAIL_BASE, 4)

    plsc.subcore_barrier()

    def _dump(r0, nrows):
        pltpu.sync_copy(acc_sh.at[pl.ds(r0, nrows)], rows.at[pl.ds(0, nrows)])
        pltpu.sync_copy(rows.at[pl.ds(0, nrows)], out_hbm.at[c, pl.ds(r0, nrows)])

    @pl.when(s < 15)
    def _():
        for k in range(4):
            _dump(s * 624 + k * 128, 128)
        _dump(s * 624 + 512, 112)

    @pl.when(s == 15)
    def _():
        for k in range(5):
            _dump(9360 + k * 128, 128)


def _mm_body(x_ref, w_ref, o_ref):
    o_ref[...] = jnp.dot(x_ref[...], w_ref[...],
                         preferred_element_type=jnp.float32)


def _comb_body(p_ref, x_ref, ws_ref, b_ref, o_ref):
    acc = p_ref[0] + p_ref[1]
    acc = acc + jnp.dot(x_ref[...], ws_ref[...],
                        preferred_element_type=jnp.float32)
    o_ref[...] = jnp.maximum(acc + b_ref[...], 0.0)


def kernel(x, edge_index, edge_type, edge_weight, W_lin, b_lin, W_self, b_self):
    src = edge_index[0]
    dst = edge_index[1]

    Wt = jnp.transpose(W_lin.reshape(_OUT, _R, _D), (2, 1, 0)).reshape(_D, _R * _OUT)

    xw = pl.pallas_call(
        _mm_body,
        grid=(10,),
        in_specs=[
            pl.BlockSpec((1000, _D), lambda i: (i, 0)),
            pl.BlockSpec((_D, _R * _OUT), lambda i: (0, 0)),
        ],
        out_specs=pl.BlockSpec((1000, _R * _OUT), lambda i: (i, 0)),
        out_shape=jax.ShapeDtypeStruct((_N, _R * _OUT), jnp.float32),
    )(x, Wt)
    xw = xw.reshape(_NR, _OUT)

    mesh = plsc.VectorSubcoreMesh(core_axis_name="c", subcore_axis_name="s",
                                  num_cores=_NC, num_subcores=_NS)
    sc_fn = pl.kernel(
        _sc_body,
        out_type=jax.ShapeDtypeStruct((_NC, _N, _OUT), jnp.float32),
        mesh=mesh,
        scratch_types=[
            pltpu.VMEM((1280,), jnp.float32),
            pltpu.VMEM((_CH,), jnp.int32),
            pltpu.VMEM((_CH,), jnp.int32),
            pltpu.VMEM((_CH,), jnp.int32),
            pltpu.VMEM((_CH,), jnp.float32),
            pltpu.VMEM((_GPC, _GRP), jnp.int32),
            pltpu.VMEM((_GPC, _GRP), jnp.int32),
            pltpu.VMEM((_GPC, _GRP), jnp.int32),
            pltpu.VMEM((_CH,), jnp.float32),
            pltpu.VMEM((_CH,), jnp.float32),
            pltpu.VMEM((_GRP, _OUT), jnp.float32),
            pltpu.VMEM((_GRP, _OUT), jnp.float32),
            pltpu.VMEM_SHARED((_N, _OUT), jnp.float32),
            pltpu.VMEM_SHARED((_NR,), jnp.float32),
            pltpu.SemaphoreType.DMA,
            pltpu.SemaphoreType.DMA,
        ],
    )
    parts = sc_fn(src, dst, edge_type, edge_weight, xw)

    bias = (b_lin + b_self).reshape(1, _OUT)
    Wst = W_self.T

    out = pl.pallas_call(
        _comb_body,
        grid=(10,),
        in_specs=[
            pl.BlockSpec((_NC, 1000, _OUT), lambda i: (0, i, 0)),
            pl.BlockSpec((1000, _D), lambda i: (i, 0)),
            pl.BlockSpec((_D, _OUT), lambda i: (0, 0)),
            pl.BlockSpec((1, _OUT), lambda i: (0, 0)),
        ],
        out_specs=pl.BlockSpec((1000, _OUT), lambda i: (i, 0)),
        out_shape=jax.ShapeDtypeStruct((_N, _OUT), jnp.float32),
    )(parts, x, Wst, bias)
    return out
